# baseline (speedup 1.0000x reference)
import jax
import jax.numpy as jnp
from jax.experimental import pallas as pl

NUM_Q = 8
K = 1024
D = 128
TT = 2304


def _prep_body(embed_ref, hi2_ref, lo_ref, lo2_ref, n_ref):
    for i in range(NUM_Q):
        tab = embed_ref[i]
        hi = tab.astype(jnp.bfloat16)
        hi2_ref[i] = hi + hi
        hi_f32 = hi.astype(jnp.float32)
        lo_f32 = tab - hi_f32
        lo = lo_f32.astype(jnp.bfloat16)
        lo_ref[i] = lo
        lo2_ref[i] = (lo_f32 - lo.astype(jnp.float32)).astype(jnp.bfloat16)
        n = jnp.sum(tab * tab, axis=1, keepdims=True)
        n_ref[i] = jnp.broadcast_to(n, (K, D))


def _stage(resid, hi2, lo, lo2, n):
    scaled = jnp.sum(resid * resid, axis=0, keepdims=True)
    g2 = jax.lax.dot_general(
        hi2, resid.astype(jnp.bfloat16), (((1,), (0,)), ((), ())),
        preferred_element_type=jnp.float32)
    x = (scaled - g2) + n
    idx = jnp.argmin(x, axis=0).astype(jnp.int32)
    onehot = (jax.lax.broadcasted_iota(jnp.int32, (K, TT), 0)
              == idx[None, :]).astype(jnp.bfloat16)
    d1 = jax.lax.dot_general(
        hi2, onehot, (((0,), (0,)), ((), ())),
        preferred_element_type=jnp.float32)
    d2 = jax.lax.dot_general(
        lo, onehot, (((0,), (0,)), ((), ())),
        preferred_element_type=jnp.float32)
    d3 = jax.lax.dot_general(
        lo2, onehot, (((0,), (0,)), ((), ())),
        preferred_element_type=jnp.float32)
    return idx, resid - ((0.5 * d1 + d2) + d3)


def _rvq_body(emb_ref, hi2_ref, lo_ref, lo2_ref, n_ref, outa_ref, outb_ref):
    ra = emb_ref[0]
    rb = emb_ref[1]
    for i in range(NUM_Q):
        hi2 = hi2_ref[i]
        lo = lo_ref[i]
        lo2 = lo2_ref[i]
        n = n_ref[i][:, 0:1]
        ia, ra = _stage(ra, hi2, lo, lo2, n)
        ib, rb = _stage(rb, hi2, lo, lo2, n)
        outa_ref[i, :] = ia
        outb_ref[i, :] = ib


def kernel(embeddings, embed):
    b, d, t = embeddings.shape
    num_q, k, _ = embed.shape
    t_tiles = (t + TT - 1) // TT
    t_pad = t_tiles * TT

    tbl_ty = jax.ShapeDtypeStruct((num_q, k, d), jnp.bfloat16)
    hi2, lo, lo2, n_bc = pl.pallas_call(
        _prep_body,
        out_shape=(tbl_ty, tbl_ty, tbl_ty,
                   jax.ShapeDtypeStruct((num_q, k, d), jnp.float32)),
    )(embed)

    full = lambda bi, ti: (0, 0, 0)
    half = b // 2
    idx_ty = jax.ShapeDtypeStruct((num_q, half * t_pad), jnp.int32)
    out_spec = pl.BlockSpec((num_q, TT), lambda bi, ti: (0, bi * t_tiles + ti))
    outa, outb = pl.pallas_call(
        _rvq_body,
        grid=(half, t_tiles),
        in_specs=[
            pl.BlockSpec((2, d, TT), lambda bi, ti: (bi, 0, ti)),
            pl.BlockSpec((num_q, k, d), full),
            pl.BlockSpec((num_q, k, d), full),
            pl.BlockSpec((num_q, k, d), full),
            pl.BlockSpec((num_q, k, d), full),
        ],
        out_specs=(out_spec, out_spec),
        out_shape=(idx_ty, idx_ty),
    )(embeddings, hi2, lo, lo2, n_bc)
    outa = outa.reshape(num_q, half, 1, t_pad)
    outb = outb.reshape(num_q, half, 1, t_pad)
    out = jnp.concatenate([outa, outb], axis=2).reshape(num_q, b, t_pad)
    return out[:, :, :t]


# speedup vs baseline: 1.1896x; 1.1896x over previous
import jax
import jax.numpy as jnp
from jax.experimental import pallas as pl
from jax.experimental.pallas import tpu as pltpu

NUM_Q = 8
K = 1024
D = 128
TT = 768


def _prep_body(embed_ref, hi2_ref, lo_ref, lo2_ref, n_ref):
    for i in range(NUM_Q):
        tab = embed_ref[i]
        hi = tab.astype(jnp.bfloat16)
        hi2_ref[i] = hi + hi
        hi_f32 = hi.astype(jnp.float32)
        lo_f32 = tab - hi_f32
        lo = lo_f32.astype(jnp.bfloat16)
        lo_ref[i] = lo
        lo2_ref[i] = (lo_f32 - lo.astype(jnp.float32)).astype(jnp.bfloat16)
        n = jnp.sum(tab * tab, axis=1, keepdims=True)
        n_ref[i] = jnp.broadcast_to(n, (K, D))


def _stage(resid, hi2, lo, lo2, n):
    scaled = jnp.sum(resid * resid, axis=0, keepdims=True)
    g2 = jax.lax.dot_general(
        hi2, resid.astype(jnp.bfloat16), (((1,), (0,)), ((), ())),
        preferred_element_type=jnp.float32)
    x = (scaled - g2) + n
    idx = jnp.argmin(x, axis=0).astype(jnp.int32)
    onehot = (jax.lax.broadcasted_iota(jnp.int32, (K, TT), 0)
              == idx[None, :]).astype(jnp.bfloat16)
    d1 = jax.lax.dot_general(
        hi2, onehot, (((0,), (0,)), ((), ())),
        preferred_element_type=jnp.float32)
    d2 = jax.lax.dot_general(
        lo, onehot, (((0,), (0,)), ((), ())),
        preferred_element_type=jnp.float32)
    d3 = jax.lax.dot_general(
        lo2, onehot, (((0,), (0,)), ((), ())),
        preferred_element_type=jnp.float32)
    return idx, resid - ((0.5 * d1 + d2) + d3)


def _rvq_body(emb_ref, embed_ref, outa_ref, outb_ref,
              hi2_ref, lo_ref, lo2_ref, n_ref):
    @pl.when(jnp.logical_and(pl.program_id(0) == 0, pl.program_id(1) == 0))
    def _():
        _prep_body(embed_ref, hi2_ref, lo_ref, lo2_ref, n_ref)

    ra = emb_ref[0]
    rb = emb_ref[1]
    for i in range(NUM_Q):
        hi2 = hi2_ref[i]
        lo = lo_ref[i]
        lo2 = lo2_ref[i]
        n = n_ref[i][:, 0:1]
        ia, ra = _stage(ra, hi2, lo, lo2, n)
        ib, rb = _stage(rb, hi2, lo, lo2, n)
        outa_ref[i, :] = ia
        outb_ref[i, :] = ib


def kernel(embeddings, embed):
    b, d, t = embeddings.shape
    num_q, k, _ = embed.shape
    t_tiles = (t + TT - 1) // TT
    t_pad = t_tiles * TT

    full = lambda bi, ti: (0, 0, 0)
    half = b // 2
    idx_ty = jax.ShapeDtypeStruct((num_q, half * t_pad), jnp.int32)
    out_spec = pl.BlockSpec((num_q, TT), lambda bi, ti: (0, bi * t_tiles + ti))
    outa, outb = pl.pallas_call(
        _rvq_body,
        grid=(half, t_tiles),
        in_specs=[
            pl.BlockSpec((2, d, TT), lambda bi, ti: (bi, 0, ti)),
            pl.BlockSpec((num_q, k, d), full),
        ],
        out_specs=(out_spec, out_spec),
        out_shape=(idx_ty, idx_ty),
        scratch_shapes=[
            pltpu.VMEM((num_q, k, d), jnp.bfloat16),
            pltpu.VMEM((num_q, k, d), jnp.bfloat16),
            pltpu.VMEM((num_q, k, d), jnp.bfloat16),
            pltpu.VMEM((num_q, k, d), jnp.float32),
        ],
    )(embeddings, embed)
    outa = outa.reshape(num_q, half, 1, t_pad)
    outb = outb.reshape(num_q, half, 1, t_pad)
    out = jnp.concatenate([outa, outb], axis=2).reshape(num_q, b, t_pad)
    return out[:, :, :t]
